# baseline (speedup 1.0000x reference)
import functools

import jax
import jax.numpy as jnp
from jax import lax
from jax.experimental import pallas as pl
from jax.experimental.pallas import tpu as pltpu
from jax.experimental.pallas import tpu_sc as plsc

BATCH = 16384
EMB_DIM = 64
PAIR = 2 * EMB_DIM
NUM_CORES = 2
NUM_SUBCORES = 16
NUM_WORKERS = NUM_CORES * NUM_SUBCORES
ROWS_PER_W = BATCH // NUM_WORKERS
CHUNK = 128
N_CHUNKS = ROWS_PER_W // CHUNK
LANES = 16


@functools.cache
def _sc_gather_kernel():
    mesh = plsc.VectorSubcoreMesh(core_axis_name="c", subcore_axis_name="s")

    @functools.partial(
        pl.kernel,
        mesh=mesh,
        out_type=[
            jax.ShapeDtypeStruct((BATCH, PAIR), jnp.int32),
            jax.ShapeDtypeStruct((BATCH, PAIR), jnp.int32),
            jax.ShapeDtypeStruct((BATCH,), jnp.float32),
            jax.ShapeDtypeStruct((BATCH,), jnp.float32),
        ],
        scratch_types=[
            pltpu.VMEM((ROWS_PER_W,), jnp.int32),
            pltpu.VMEM((ROWS_PER_W,), jnp.int32),
            pltpu.VMEM((ROWS_PER_W,), jnp.float32),
            pltpu.VMEM((CHUNK, PAIR), jnp.int32),
            pltpu.SemaphoreType.DMA,
        ],
    )
    def _sc_gather(uids_hbm, iids_hbm, utab_hbm, itab_hbm,
                   uout_hbm, iout_hbm, uflag_hbm, iflag_hbm,
                   ids_v, tid_v, flag_v, tiles_v, sem):
        wid = lax.axis_index("s") * NUM_CORES + lax.axis_index("c")
        base = wid * ROWS_PER_W

        def one_table(ids_hbm, tab_hbm, out_hbm, flag_hbm):
            pltpu.sync_copy(ids_hbm.at[pl.ds(base, ROWS_PER_W)], ids_v)
            for k in range(ROWS_PER_W // LANES):
                ids = ids_v[pl.ds(k * LANES, LANES)]
                tid_v[pl.ds(k * LANES, LANES)] = (
                    ((ids >> BLK_BITS) << Q_BITS) | (ids & (QROWS - 1)))
                flag_v[pl.ds(k * LANES, LANES)] = (
                    (ids >> Q_BITS) & 3).astype(jnp.float32)
            for j in range(N_CHUNKS):
                pltpu.async_copy(
                    tab_hbm.at[tid_v.at[pl.ds(j * CHUNK, CHUNK)]],
                    tiles_v, sem).wait()
                pltpu.sync_copy(
                    tiles_v, out_hbm.at[pl.ds(base + j * CHUNK, CHUNK)])
            pltpu.sync_copy(flag_v, flag_hbm.at[pl.ds(base, ROWS_PER_W)])

        one_table(uids_hbm, utab_hbm, uout_hbm, uflag_hbm)
        one_table(iids_hbm, itab_hbm, iout_hbm, iflag_hbm)

    return _sc_gather


_RP_LANES = 16384
QROWS = _RP_LANES // 4
HALF = EMB_DIM // 2
_RP_GRID = -(-1000000 // _RP_LANES)
TAB_ROWS = _RP_GRID * QROWS
BLK_BITS = _RP_LANES.bit_length() - 1
Q_BITS = QROWS.bit_length() - 1


_MASK_HI = 0xFFFF0000 - 0x100000000


def _repack_body(t_ref, o_ref):
    t = lax.bitcast_convert_type(t_ref[...], jnp.int32)
    hi = t[HALF:] & jnp.int32(_MASK_HI)
    lo = lax.bitcast_convert_type(
        lax.shift_right_logical(
            lax.bitcast_convert_type(t[:HALF], jnp.uint32),
            jnp.uint32(16)), jnp.int32)
    packed = (hi | lo).T
    o_ref[...] = jnp.concatenate(
        [packed[:QROWS], packed[QROWS:2 * QROWS],
         packed[2 * QROWS:3 * QROWS], packed[3 * QROWS:]], axis=1)


def _repack(tabT):
    return pl.pallas_call(
        _repack_body,
        grid=(_RP_GRID,),
        in_specs=[pl.BlockSpec((EMB_DIM, _RP_LANES), lambda g: (0, g))],
        out_specs=pl.BlockSpec((QROWS, PAIR), lambda g: (g, 0)),
        out_shape=jax.ShapeDtypeStruct((TAB_ROWS, PAIR), jnp.int32),
    )(tabT)


def _tc_body(u_ref, i_ref, uf_ref, if_ref, wu_ref, bu_ref, wi_ref, bi_ref,
             o_ref):
    dn = (((1,), (1,)), ((), ()))

    def unquad(x, q):
        quarter = jnp.where(
            q < 1.5,
            jnp.where(q < 0.5, x[:, :HALF], x[:, HALF:2 * HALF]),
            jnp.where(q < 2.5, x[:, 2 * HALF:3 * HALF], x[:, 3 * HALF:]))
        lo = lax.bitcast_convert_type(
            lax.shift_left(quarter, jnp.int32(16)), jnp.float32)
        hi = lax.bitcast_convert_type(quarter & jnp.int32(_MASK_HI), jnp.float32)
        return jnp.concatenate([lo, hi], axis=1)

    u_row = unquad(u_ref[...], uf_ref[...])
    i_row = unquad(i_ref[...], if_ref[...])
    u = lax.dot_general(u_row, wu_ref[...], dn,
                        preferred_element_type=jnp.float32) + bu_ref[...]
    u = jnp.maximum(u, 0.0)
    i = lax.dot_general(i_row, wi_ref[...], dn,
                        preferred_element_type=jnp.float32) + bi_ref[...]
    i = jnp.maximum(i, 0.0)
    un = jnp.sqrt(jnp.sum(u * u, axis=1, keepdims=True))
    inn = jnp.sqrt(jnp.sum(i * i, axis=1, keepdims=True))
    denom = jnp.maximum(un, 1e-12) * jnp.maximum(inn, 1e-12)
    o_ref[...] = jnp.sum(u * i, axis=1, keepdims=True) / denom


_TC_BLOCK = 2048


def _tc_scores(u_rows, i_rows, uf, if_, Wu, bu2, Wi, bi2):
    grid = (BATCH // _TC_BLOCK,)
    return pl.pallas_call(
        _tc_body,
        grid=grid,
        in_specs=[
            pl.BlockSpec((_TC_BLOCK, PAIR), lambda g: (g, 0)),
            pl.BlockSpec((_TC_BLOCK, PAIR), lambda g: (g, 0)),
            pl.BlockSpec((_TC_BLOCK, 1), lambda g: (g, 0)),
            pl.BlockSpec((_TC_BLOCK, 1), lambda g: (g, 0)),
            pl.BlockSpec((EMB_DIM, EMB_DIM), lambda g: (0, 0)),
            pl.BlockSpec((1, EMB_DIM), lambda g: (0, 0)),
            pl.BlockSpec((EMB_DIM, EMB_DIM), lambda g: (0, 0)),
            pl.BlockSpec((1, EMB_DIM), lambda g: (0, 0)),
        ],
        out_specs=pl.BlockSpec((_TC_BLOCK, 1), lambda g: (g, 0)),
        out_shape=jax.ShapeDtypeStruct((BATCH, 1), jnp.float32),
    )(u_rows, i_rows, uf, if_, Wu, bu2, Wi, bi2)


def kernel(user_ids, item_ids, user_emb, item_emb, Wu, bu, Wi, bi):
    uids = user_ids.astype(jnp.int32)
    iids = item_ids.astype(jnp.int32)
    utab = _repack(user_emb.T)
    itab = _repack(item_emb.T)
    u_rows, i_rows, uf, if_ = _sc_gather_kernel()(uids, iids, utab, itab)
    scores = _tc_scores(u_rows, i_rows, uf.reshape(BATCH, 1),
                        if_.reshape(BATCH, 1), Wu, bu.reshape(1, EMB_DIM),
                        Wi, bi.reshape(1, EMB_DIM))
    return scores.reshape(BATCH)


# speedup vs baseline: 1.0060x; 1.0060x over previous
import functools

import jax
import jax.numpy as jnp
from jax import lax
from jax.experimental import pallas as pl
from jax.experimental.pallas import tpu as pltpu
from jax.experimental.pallas import tpu_sc as plsc

BATCH = 16384
EMB_DIM = 64
PAIR = 2 * EMB_DIM
NUM_CORES = 2
NUM_SUBCORES = 16
NUM_WORKERS = NUM_CORES * NUM_SUBCORES
ROWS_PER_W = BATCH // NUM_WORKERS
CHUNK = 128
N_CHUNKS = ROWS_PER_W // CHUNK
LANES = 16


@functools.cache
def _sc_gather_kernel():
    mesh = plsc.VectorSubcoreMesh(core_axis_name="c", subcore_axis_name="s")

    @functools.partial(
        pl.kernel,
        mesh=mesh,
        out_type=[
            jax.ShapeDtypeStruct((BATCH, PAIR), jnp.int32),
            jax.ShapeDtypeStruct((BATCH, PAIR), jnp.int32),
            jax.ShapeDtypeStruct((BATCH,), jnp.float32),
            jax.ShapeDtypeStruct((BATCH,), jnp.float32),
        ],
        scratch_types=[
            pltpu.VMEM((ROWS_PER_W,), jnp.int32),
            pltpu.VMEM((ROWS_PER_W,), jnp.int32),
            pltpu.VMEM((ROWS_PER_W,), jnp.float32),
            pltpu.VMEM((CHUNK, PAIR), jnp.int32),
            pltpu.SemaphoreType.DMA,
        ],
    )
    def _sc_gather(uids_hbm, iids_hbm, utab_hbm, itab_hbm,
                   uout_hbm, iout_hbm, uflag_hbm, iflag_hbm,
                   ids_v, tid_v, flag_v, tiles_v, sem):
        wid = lax.axis_index("s") * NUM_CORES + lax.axis_index("c")
        base = wid * ROWS_PER_W

        def one_table(ids_hbm, tab_hbm, out_hbm, flag_hbm):
            pltpu.sync_copy(ids_hbm.at[pl.ds(base, ROWS_PER_W)], ids_v)
            for k in range(ROWS_PER_W // LANES):
                ids = ids_v[pl.ds(k * LANES, LANES)]
                tid_v[pl.ds(k * LANES, LANES)] = (
                    ((ids >> BLK_BITS) << Q_BITS) | (ids & (QROWS - 1)))
                flag_v[pl.ds(k * LANES, LANES)] = (
                    (ids >> Q_BITS) & 3).astype(jnp.float32)
            for j in range(N_CHUNKS):
                pltpu.async_copy(
                    tab_hbm.at[tid_v.at[pl.ds(j * CHUNK, CHUNK)]],
                    tiles_v, sem).wait()
                pltpu.sync_copy(
                    tiles_v, out_hbm.at[pl.ds(base + j * CHUNK, CHUNK)])
            pltpu.sync_copy(flag_v, flag_hbm.at[pl.ds(base, ROWS_PER_W)])

        one_table(uids_hbm, utab_hbm, uout_hbm, uflag_hbm)
        one_table(iids_hbm, itab_hbm, iout_hbm, iflag_hbm)

    return _sc_gather


_RP_LANES = 32768
QROWS = _RP_LANES // 4
HALF = EMB_DIM // 2
_RP_GRID = -(-1000000 // _RP_LANES)
TAB_ROWS = _RP_GRID * QROWS
BLK_BITS = _RP_LANES.bit_length() - 1
Q_BITS = QROWS.bit_length() - 1


_MASK_HI = 0xFFFF0000 - 0x100000000


def _repack_body(t_ref, o_ref):
    t = lax.bitcast_convert_type(t_ref[...], jnp.int32)
    hi = t[HALF:] & jnp.int32(_MASK_HI)
    lo = lax.bitcast_convert_type(
        lax.shift_right_logical(
            lax.bitcast_convert_type(t[:HALF], jnp.uint32),
            jnp.uint32(16)), jnp.int32)
    packed = (hi | lo).T
    o_ref[...] = jnp.concatenate(
        [packed[:QROWS], packed[QROWS:2 * QROWS],
         packed[2 * QROWS:3 * QROWS], packed[3 * QROWS:]], axis=1)


def _repack(tabT):
    return pl.pallas_call(
        _repack_body,
        grid=(_RP_GRID,),
        in_specs=[pl.BlockSpec((EMB_DIM, _RP_LANES), lambda g: (0, g))],
        out_specs=pl.BlockSpec((QROWS, PAIR), lambda g: (g, 0)),
        out_shape=jax.ShapeDtypeStruct((TAB_ROWS, PAIR), jnp.int32),
    )(tabT)


def _tc_body(u_ref, i_ref, uf_ref, if_ref, wu_ref, bu_ref, wi_ref, bi_ref,
             o_ref):
    dn = (((1,), (1,)), ((), ()))

    def unquad(x, q):
        quarter = jnp.where(
            q < 1.5,
            jnp.where(q < 0.5, x[:, :HALF], x[:, HALF:2 * HALF]),
            jnp.where(q < 2.5, x[:, 2 * HALF:3 * HALF], x[:, 3 * HALF:]))
        lo = lax.bitcast_convert_type(
            lax.shift_left(quarter, jnp.int32(16)), jnp.float32)
        hi = lax.bitcast_convert_type(quarter & jnp.int32(_MASK_HI), jnp.float32)
        return jnp.concatenate([lo, hi], axis=1)

    u_row = unquad(u_ref[...], uf_ref[...])
    i_row = unquad(i_ref[...], if_ref[...])
    u = lax.dot_general(u_row, wu_ref[...], dn,
                        preferred_element_type=jnp.float32) + bu_ref[...]
    u = jnp.maximum(u, 0.0)
    i = lax.dot_general(i_row, wi_ref[...], dn,
                        preferred_element_type=jnp.float32) + bi_ref[...]
    i = jnp.maximum(i, 0.0)
    un = jnp.sqrt(jnp.sum(u * u, axis=1, keepdims=True))
    inn = jnp.sqrt(jnp.sum(i * i, axis=1, keepdims=True))
    denom = jnp.maximum(un, 1e-12) * jnp.maximum(inn, 1e-12)
    o_ref[...] = jnp.sum(u * i, axis=1, keepdims=True) / denom


_TC_BLOCK = 2048


def _tc_scores(u_rows, i_rows, uf, if_, Wu, bu2, Wi, bi2):
    grid = (BATCH // _TC_BLOCK,)
    return pl.pallas_call(
        _tc_body,
        grid=grid,
        in_specs=[
            pl.BlockSpec((_TC_BLOCK, PAIR), lambda g: (g, 0)),
            pl.BlockSpec((_TC_BLOCK, PAIR), lambda g: (g, 0)),
            pl.BlockSpec((_TC_BLOCK, 1), lambda g: (g, 0)),
            pl.BlockSpec((_TC_BLOCK, 1), lambda g: (g, 0)),
            pl.BlockSpec((EMB_DIM, EMB_DIM), lambda g: (0, 0)),
            pl.BlockSpec((1, EMB_DIM), lambda g: (0, 0)),
            pl.BlockSpec((EMB_DIM, EMB_DIM), lambda g: (0, 0)),
            pl.BlockSpec((1, EMB_DIM), lambda g: (0, 0)),
        ],
        out_specs=pl.BlockSpec((_TC_BLOCK, 1), lambda g: (g, 0)),
        out_shape=jax.ShapeDtypeStruct((BATCH, 1), jnp.float32),
    )(u_rows, i_rows, uf, if_, Wu, bu2, Wi, bi2)


def kernel(user_ids, item_ids, user_emb, item_emb, Wu, bu, Wi, bi):
    uids = user_ids.astype(jnp.int32)
    iids = item_ids.astype(jnp.int32)
    utab = _repack(user_emb.T)
    itab = _repack(item_emb.T)
    u_rows, i_rows, uf, if_ = _sc_gather_kernel()(uids, iids, utab, itab)
    scores = _tc_scores(u_rows, i_rows, uf.reshape(BATCH, 1),
                        if_.reshape(BATCH, 1), Wu, bu.reshape(1, EMB_DIM),
                        Wi, bi.reshape(1, EMB_DIM))
    return scores.reshape(BATCH)


# speedup vs baseline: 1.0250x; 1.0189x over previous
import functools

import jax
import jax.numpy as jnp
from jax import lax
from jax.experimental import pallas as pl
from jax.experimental.pallas import tpu as pltpu
from jax.experimental.pallas import tpu_sc as plsc

BATCH = 16384
EMB_DIM = 64
PAIR = 2 * EMB_DIM
NUM_CORES = 2
NUM_SUBCORES = 16
NUM_WORKERS = NUM_CORES * NUM_SUBCORES
ROWS_PER_W = BATCH // NUM_WORKERS
CHUNK = 128
N_CHUNKS = ROWS_PER_W // CHUNK
LANES = 16


@functools.cache
def _sc_gather_kernel():
    mesh = plsc.VectorSubcoreMesh(core_axis_name="c", subcore_axis_name="s")

    @functools.partial(
        pl.kernel,
        mesh=mesh,
        out_type=[
            jax.ShapeDtypeStruct((BATCH, PAIR), jnp.int32),
            jax.ShapeDtypeStruct((BATCH,), jnp.float32),
        ],
        scratch_types=[
            pltpu.VMEM((ROWS_PER_W,), jnp.int32),
            pltpu.VMEM((ROWS_PER_W,), jnp.int32),
            pltpu.VMEM((ROWS_PER_W,), jnp.float32),
            pltpu.VMEM((CHUNK, PAIR), jnp.int32),
            pltpu.SemaphoreType.DMA,
        ],
    )
    def _sc_gather(ids_hbm, tab_hbm, out_hbm, flag_hbm,
                   ids_v, tid_v, flag_v, tiles_v, sem):
        wid = lax.axis_index("s") * NUM_CORES + lax.axis_index("c")
        base = wid * ROWS_PER_W
        pltpu.sync_copy(ids_hbm.at[pl.ds(base, ROWS_PER_W)], ids_v)
        for k in range(ROWS_PER_W // LANES):
            ids = ids_v[pl.ds(k * LANES, LANES)]
            tid_v[pl.ds(k * LANES, LANES)] = (
                ((ids >> BLK_BITS) << Q_BITS) | (ids & (QROWS - 1)))
            flag_v[pl.ds(k * LANES, LANES)] = (
                (ids >> Q_BITS) & 3).astype(jnp.float32)
        for j in range(N_CHUNKS):
            pltpu.async_copy(
                tab_hbm.at[tid_v.at[pl.ds(j * CHUNK, CHUNK)]],
                tiles_v, sem).wait()
            pltpu.sync_copy(
                tiles_v, out_hbm.at[pl.ds(base + j * CHUNK, CHUNK)])
        pltpu.sync_copy(flag_v, flag_hbm.at[pl.ds(base, ROWS_PER_W)])

    return _sc_gather


_RP_LANES = 32768
QROWS = _RP_LANES // 4
HALF = EMB_DIM // 2
_RP_GRID = -(-1000000 // _RP_LANES)
TAB_ROWS = _RP_GRID * QROWS
BLK_BITS = _RP_LANES.bit_length() - 1
Q_BITS = QROWS.bit_length() - 1


_MASK_HI = 0xFFFF0000 - 0x100000000


def _repack_body(t_ref, o_ref):
    t = lax.bitcast_convert_type(t_ref[...], jnp.int32)
    hi = t[HALF:] & jnp.int32(_MASK_HI)
    lo = lax.bitcast_convert_type(
        lax.shift_right_logical(
            lax.bitcast_convert_type(t[:HALF], jnp.uint32),
            jnp.uint32(16)), jnp.int32)
    packed = (hi | lo).T
    o_ref[...] = jnp.concatenate(
        [packed[:QROWS], packed[QROWS:2 * QROWS],
         packed[2 * QROWS:3 * QROWS], packed[3 * QROWS:]], axis=1)


def _repack(tabT):
    return pl.pallas_call(
        _repack_body,
        grid=(_RP_GRID,),
        in_specs=[pl.BlockSpec((EMB_DIM, _RP_LANES), lambda g: (0, g))],
        out_specs=pl.BlockSpec((QROWS, PAIR), lambda g: (g, 0)),
        out_shape=jax.ShapeDtypeStruct((TAB_ROWS, PAIR), jnp.int32),
    )(tabT)


def _tc_body(u_ref, i_ref, uf_ref, if_ref, wu_ref, bu_ref, wi_ref, bi_ref,
             o_ref):
    dn = (((1,), (1,)), ((), ()))

    def unquad(x, q):
        quarter = jnp.where(
            q < 1.5,
            jnp.where(q < 0.5, x[:, :HALF], x[:, HALF:2 * HALF]),
            jnp.where(q < 2.5, x[:, 2 * HALF:3 * HALF], x[:, 3 * HALF:]))
        lo = lax.bitcast_convert_type(
            lax.shift_left(quarter, jnp.int32(16)), jnp.float32)
        hi = lax.bitcast_convert_type(quarter & jnp.int32(_MASK_HI), jnp.float32)
        return jnp.concatenate([lo, hi], axis=1)

    u_row = unquad(u_ref[...], uf_ref[...])
    i_row = unquad(i_ref[...], if_ref[...])
    u = lax.dot_general(u_row, wu_ref[...], dn,
                        preferred_element_type=jnp.float32) + bu_ref[...]
    u = jnp.maximum(u, 0.0)
    i = lax.dot_general(i_row, wi_ref[...], dn,
                        preferred_element_type=jnp.float32) + bi_ref[...]
    i = jnp.maximum(i, 0.0)
    un = jnp.sqrt(jnp.sum(u * u, axis=1, keepdims=True))
    inn = jnp.sqrt(jnp.sum(i * i, axis=1, keepdims=True))
    denom = jnp.maximum(un, 1e-12) * jnp.maximum(inn, 1e-12)
    o_ref[...] = jnp.sum(u * i, axis=1, keepdims=True) / denom


_TC_BLOCK = 2048


def _tc_scores(u_rows, i_rows, uf, if_, Wu, bu2, Wi, bi2):
    grid = (BATCH // _TC_BLOCK,)
    return pl.pallas_call(
        _tc_body,
        grid=grid,
        in_specs=[
            pl.BlockSpec((_TC_BLOCK, PAIR), lambda g: (g, 0)),
            pl.BlockSpec((_TC_BLOCK, PAIR), lambda g: (g, 0)),
            pl.BlockSpec((_TC_BLOCK, 1), lambda g: (g, 0)),
            pl.BlockSpec((_TC_BLOCK, 1), lambda g: (g, 0)),
            pl.BlockSpec((EMB_DIM, EMB_DIM), lambda g: (0, 0)),
            pl.BlockSpec((1, EMB_DIM), lambda g: (0, 0)),
            pl.BlockSpec((EMB_DIM, EMB_DIM), lambda g: (0, 0)),
            pl.BlockSpec((1, EMB_DIM), lambda g: (0, 0)),
        ],
        out_specs=pl.BlockSpec((_TC_BLOCK, 1), lambda g: (g, 0)),
        out_shape=jax.ShapeDtypeStruct((BATCH, 1), jnp.float32),
    )(u_rows, i_rows, uf, if_, Wu, bu2, Wi, bi2)


def kernel(user_ids, item_ids, user_emb, item_emb, Wu, bu, Wi, bi):
    uids = user_ids.astype(jnp.int32)
    iids = item_ids.astype(jnp.int32)
    gather = _sc_gather_kernel()
    utab = _repack(user_emb.T)
    u_rows, uf = gather(uids, utab)
    itab = _repack(item_emb.T)
    i_rows, if_ = gather(iids, itab)
    scores = _tc_scores(u_rows, i_rows, uf.reshape(BATCH, 1),
                        if_.reshape(BATCH, 1), Wu, bu.reshape(1, EMB_DIM),
                        Wi, bi.reshape(1, EMB_DIM))
    return scores.reshape(BATCH)
